# baseline (speedup 1.0000x reference)
import functools

import ml_dtypes
import numpy as np

import jax
import jax.numpy as jnp
from jax.experimental import pallas as pl
from jax.experimental.pallas import tpu as pltpu

_NUM_FILTERS = 4
_NUM_FC = 32
_NUM_CLASSES = 10
_IMG = 28
_KSIZE = 5
_CONV_OUT = _IMG - _KSIZE + 1
_POOL_OUT = _CONV_OUT // 2
_PIX = _IMG * _IMG
_PQ = _POOL_OUT * _POOL_OUT
_FEAT = _PQ * _NUM_FILTERS
_BATCH_TILE = 512

_SEL = np.zeros((_KSIZE, 2, _POOL_OUT, _IMG), np.float32)
for _k in range(_KSIZE):
    for _d in range(2):
        for _p in range(_POOL_OUT):
            _SEL[_k, _d, _p, 2 * _p + _d + _k] = 1.0

_SM = np.einsum('adpi,bcqj->abdcpqij', _SEL, _SEL).reshape(
    _KSIZE * _KSIZE, 4 * _PQ * _PIX).astype(ml_dtypes.bfloat16)


def _net_kernel(x_ref, a_ref, bp_ref, w1_ref, b1_ref, w2_ref, b2_ref, o_ref):
    z = jnp.dot(a_ref[...], x_ref[...], preferred_element_type=jnp.float32)
    blocks = []
    for f in range(_NUM_FILTERS):
        base = f * 4 * _PQ
        blocks.append(jnp.maximum(
            jnp.maximum(z[base + 0 * _PQ:base + 1 * _PQ],
                        z[base + 1 * _PQ:base + 2 * _PQ]),
            jnp.maximum(z[base + 2 * _PQ:base + 3 * _PQ],
                        z[base + 3 * _PQ:base + 4 * _PQ])))
    pooled = jnp.concatenate(blocks, axis=0)
    h = jnp.maximum(pooled + bp_ref[...], 0.0)
    h1 = jnp.dot(w1_ref[...], h, preferred_element_type=jnp.float32)
    h1 = jnp.maximum(h1 + b1_ref[...], 0.0)
    z2 = jnp.dot(w2_ref[...], h1,
                 preferred_element_type=jnp.float32) + b2_ref[...]
    m = jnp.max(z2, axis=0, keepdims=True)
    lse = jnp.log(jnp.sum(jnp.exp(z2 - m), axis=0, keepdims=True)) + m
    o_ref[...] = z2 - lse


@functools.partial(jax.jit, static_argnames=("batch_tile",))
def _forward(x, wconv, bconv, w1, b1, w2, b2, batch_tile=_BATCH_TILE):
    batch = x.shape[0]
    padded = ((batch + batch_tile - 1) // batch_tile) * batch_tile
    img = x.astype(jnp.float32)[:, 0]
    if padded != batch:
        img = jnp.pad(img, ((0, padded - batch), (0, 0), (0, 0)))
    xt = jnp.transpose(img.astype(jnp.bfloat16), (1, 2, 0)).reshape(_PIX, padded)

    a = jnp.dot(wconv.reshape(_NUM_FILTERS, _KSIZE * _KSIZE)
                .astype(jnp.bfloat16), _SM,
                preferred_element_type=jnp.bfloat16
                ).reshape(_NUM_FILTERS * 4 * _PQ, _PIX)

    bp = jnp.repeat(bconv, _PQ).reshape(_FEAT, 1)
    w1g = w1.transpose()
    b1c = b1.reshape(_NUM_FC, 1)
    w2g = w2.transpose()
    b2c = b2.reshape(_NUM_CLASSES, 1)

    out = pl.pallas_call(
        _net_kernel,
        out_shape=jax.ShapeDtypeStruct((_NUM_CLASSES, padded), jnp.float32),
        grid=(padded // batch_tile,),
        in_specs=[
            pl.BlockSpec((_PIX, batch_tile), lambda i: (0, i)),
            pl.BlockSpec((4 * _FEAT, _PIX), lambda i: (0, 0)),
            pl.BlockSpec((_FEAT, 1), lambda i: (0, 0)),
            pl.BlockSpec((_NUM_FC, _FEAT), lambda i: (0, 0)),
            pl.BlockSpec((_NUM_FC, 1), lambda i: (0, 0)),
            pl.BlockSpec((_NUM_CLASSES, _NUM_FC), lambda i: (0, 0)),
            pl.BlockSpec((_NUM_CLASSES, 1), lambda i: (0, 0)),
        ],
        out_specs=pl.BlockSpec((_NUM_CLASSES, batch_tile), lambda i: (0, i)),
        compiler_params=pltpu.CompilerParams(
            dimension_semantics=("arbitrary",)),
    )(xt, a, bp, w1g, b1c, w2g, b2c)

    return jnp.transpose(out)[:batch]


def kernel(x, wconv, bconv, w1, b1, w2, b2):
    return _forward(x, wconv, bconv, w1, b1, w2, b2)


# speedup vs baseline: 1.1730x; 1.1730x over previous
import functools

import ml_dtypes
import numpy as np

import jax
import jax.numpy as jnp
from jax.experimental import pallas as pl
from jax.experimental.pallas import tpu as pltpu

_NUM_FILTERS = 4
_NUM_FC = 32
_NUM_CLASSES = 10
_IMG = 28
_KSIZE = 5
_CONV_OUT = _IMG - _KSIZE + 1
_POOL_OUT = _CONV_OUT // 2
_PIX = _IMG * _IMG
_PQ = _POOL_OUT * _POOL_OUT
_FEAT = _PQ * _NUM_FILTERS
_BATCH_TILE = 1024

_SEL = np.zeros((_KSIZE, 2, _POOL_OUT, _IMG), np.float32)
for _k in range(_KSIZE):
    for _d in range(2):
        for _p in range(_POOL_OUT):
            _SEL[_k, _d, _p, 2 * _p + _d + _k] = 1.0

_SM = np.einsum('adpi,bcqj->abdcpqij', _SEL, _SEL).reshape(
    _KSIZE * _KSIZE, 4 * _PQ * _PIX).astype(ml_dtypes.bfloat16)


def _net_kernel(x_ref, a_ref, bp_ref, w1_ref, b1_ref, w2_ref, b2_ref, o_ref):
    z = jnp.dot(a_ref[...], x_ref[...].astype(jnp.bfloat16),
                preferred_element_type=jnp.float32)
    blocks = []
    for f in range(_NUM_FILTERS):
        base = f * 4 * _PQ
        blocks.append(jnp.maximum(
            jnp.maximum(z[base + 0 * _PQ:base + 1 * _PQ],
                        z[base + 1 * _PQ:base + 2 * _PQ]),
            jnp.maximum(z[base + 2 * _PQ:base + 3 * _PQ],
                        z[base + 3 * _PQ:base + 4 * _PQ])))
    pooled = jnp.concatenate(blocks, axis=0)
    h = jnp.maximum(pooled + bp_ref[...], 0.0)
    h1 = jnp.dot(w1_ref[...], h, preferred_element_type=jnp.float32)
    h1 = jnp.maximum(h1 + b1_ref[...], 0.0)
    z2 = jnp.dot(w2_ref[...], h1,
                 preferred_element_type=jnp.float32) + b2_ref[...]
    m = jnp.max(z2, axis=0, keepdims=True)
    lse = jnp.log(jnp.sum(jnp.exp(z2 - m), axis=0, keepdims=True)) + m
    o_ref[...] = z2 - lse


@functools.partial(jax.jit, static_argnames=("batch_tile",))
def _forward(x, wconv, bconv, w1, b1, w2, b2, batch_tile=_BATCH_TILE):
    batch = x.shape[0]
    padded = ((batch + batch_tile - 1) // batch_tile) * batch_tile
    img = x.astype(jnp.float32)[:, 0]
    if padded != batch:
        img = jnp.pad(img, ((0, padded - batch), (0, 0), (0, 0)))
    xt = jnp.transpose(img, (1, 2, 0)).reshape(_PIX, padded)

    a = jnp.dot(wconv.reshape(_NUM_FILTERS, _KSIZE * _KSIZE)
                .astype(jnp.bfloat16), _SM,
                preferred_element_type=jnp.bfloat16
                ).reshape(_NUM_FILTERS * 4 * _PQ, _PIX)

    bp = jnp.repeat(bconv, _PQ).reshape(_FEAT, 1)
    w1g = w1.transpose()
    b1c = b1.reshape(_NUM_FC, 1)
    w2g = w2.transpose()
    b2c = b2.reshape(_NUM_CLASSES, 1)

    out = pl.pallas_call(
        _net_kernel,
        out_shape=jax.ShapeDtypeStruct((_NUM_CLASSES, padded), jnp.float32),
        grid=(padded // batch_tile,),
        in_specs=[
            pl.BlockSpec((_PIX, batch_tile), lambda i: (0, i)),
            pl.BlockSpec((4 * _FEAT, _PIX), lambda i: (0, 0)),
            pl.BlockSpec((_FEAT, 1), lambda i: (0, 0)),
            pl.BlockSpec((_NUM_FC, _FEAT), lambda i: (0, 0)),
            pl.BlockSpec((_NUM_FC, 1), lambda i: (0, 0)),
            pl.BlockSpec((_NUM_CLASSES, _NUM_FC), lambda i: (0, 0)),
            pl.BlockSpec((_NUM_CLASSES, 1), lambda i: (0, 0)),
        ],
        out_specs=pl.BlockSpec((_NUM_CLASSES, batch_tile), lambda i: (0, i)),
        compiler_params=pltpu.CompilerParams(
            dimension_semantics=("arbitrary",)),
    )(xt, a, bp, w1g, b1c, w2g, b2c)

    return jnp.transpose(out)[:batch]


def kernel(x, wconv, bconv, w1, b1, w2, b2):
    return _forward(x, wconv, bconv, w1, b1, w2, b2)
